# Initial kernel scaffold; baseline (speedup 1.0000x reference)
import jax
import jax.numpy as jnp
from jax.experimental import pallas as pl


def kernel(net_weights, flat_node2pin_start, flat_node2pin, pin2net_map, flat_net2pin):
    raise NotImplementedError("write your pallas kernel here")


# speedup vs baseline: 1588.8340x; 1588.8340x over previous
import jax
import jax.numpy as jnp
from jax import lax
from jax.experimental import pallas as pl
from jax.experimental.pallas import tpu as pltpu
from jax.experimental.pallas import tpu_sc as plsc

N_NODES = 100000
N_MOVABLE = 90000
N_PINS = 3200000
N_NETS = 500000

NC, NS = 2, 16
NW = NC * NS
PIN_PER_W = N_PINS // NW
CHUNK = 2000
N_CHUNKS = PIN_PER_W // CHUNK

NET_PAD = 512000


def _netval_body(w_ref, flo_ref, fhi_ref, v_ref):
    d = fhi_ref[...] - flo_ref[...]
    w = jnp.maximum(w_ref[...], 1.0)
    den = jnp.maximum(d - 1, 1).astype(jnp.float32)
    v_ref[...] = jnp.where(d > 1, w / den, 0.0)


def _netval(w2d, flo2d, fhi2d):
    return pl.pallas_call(
        _netval_body,
        out_shape=jax.ShapeDtypeStruct((NET_PAD // 128, 128), jnp.float32),
    )(w2d, flo2d, fhi2d)


def _gather_body(n2p_hbm, p2n_hbm, v_hbm, out_hbm, idx_v, net_v, val_v, sem):
    wid = lax.axis_index("c") * NS + lax.axis_index("s")

    def chunk(k, carry):
        base = wid * PIN_PER_W + k * CHUNK
        pltpu.sync_copy(n2p_hbm.at[pl.ds(base, CHUNK)], idx_v)
        pltpu.async_copy(p2n_hbm.at[idx_v], net_v, sem).wait()
        pltpu.async_copy(v_hbm.at[net_v], val_v, sem).wait()
        pltpu.sync_copy(val_v, out_hbm.at[pl.ds(base, CHUNK)])
        return carry

    lax.fori_loop(0, N_CHUNKS, chunk, 0)


def _build_gather():
    return pl.kernel(
        _gather_body,
        out_type=jax.ShapeDtypeStruct((N_PINS,), jnp.float32),
        mesh=plsc.VectorSubcoreMesh(
            core_axis_name="c", subcore_axis_name="s", num_cores=NC,
            num_subcores=NS,
        ),
        scratch_types=[
            pltpu.VMEM((CHUNK,), jnp.int32),
            pltpu.VMEM((CHUNK,), jnp.int32),
            pltpu.VMEM((CHUNK,), jnp.float32),
            pltpu.SemaphoreType.DMA,
        ],
    )


ROWS = N_PINS // 128
RBLK = 1000
NBLK = ROWS // RBLK


def _scan_body(x_ref, y_ref, carry_ref):
    @pl.when(pl.program_id(0) == 0)
    def _():
        carry_ref[0, 0] = 0.0

    x = x_ref[...]
    r = lax.broadcasted_iota(jnp.int32, (128, 128), 0)
    col = lax.broadcasted_iota(jnp.int32, (128, 128), 1)
    tri = (r <= col).astype(jnp.float32)
    y = jnp.dot(x, tri, preferred_element_type=jnp.float32)
    t = y[:, 127:128]
    e = jnp.concatenate([jnp.zeros((1, 1), jnp.float32), t[:-1, :]], axis=0)
    k = 1
    while k < RBLK:
        e = e + jnp.concatenate(
            [jnp.zeros((k, 1), jnp.float32), e[:-k, :]], axis=0
        )
        k *= 2
    c = carry_ref[0, 0]
    y_ref[...] = y + e + c
    carry_ref[0, 0] = c + jnp.sum(t[RBLK - 1:, :]) + jnp.sum(e[RBLK - 1:, :])


def _scan(val2d):
    return pl.pallas_call(
        _scan_body,
        grid=(NBLK,),
        in_specs=[pl.BlockSpec((RBLK, 128), lambda i: (i, 0))],
        out_specs=pl.BlockSpec((RBLK, 128), lambda i: (i, 0)),
        out_shape=jax.ShapeDtypeStruct((ROWS, 128), jnp.float32),
        scratch_shapes=[pltpu.SMEM((1, 1), jnp.float32)],
    )(val2d)


NODES_PER_W = 3136
OUT_PAD = NW * NODES_PER_W
SPAD = NODES_PER_W + 16
START_PAD = (NW - 1) * NODES_PER_W + SPAD


def _bound_body(s_hbm, st_hbm, out_hbm, sv_ref, idx_ref, g_ref, ob_ref, sem):
    wid = lax.axis_index("c") * NS + lax.axis_index("s")
    lo = wid * NODES_PER_W
    pltpu.sync_copy(st_hbm.at[pl.ds(lo, SPAD)], sv_ref)

    def mk_idx(k, carry):
        sv = sv_ref[pl.ds(k * 16, 16)]
        idx_ref[pl.ds(k * 16, 16)] = jnp.maximum(sv - 1, 0)
        return carry

    lax.fori_loop(0, SPAD // 16, mk_idx, 0)
    pltpu.async_copy(s_hbm.at[idx_ref], g_ref, sem).wait()
    iot = lax.iota(jnp.int32, 16)

    def diff(k, carry):
        sv_a = sv_ref[pl.ds(k * 16, 16)]
        sv_b = sv_ref[pl.ds(k * 16 + 1, 16)]
        g_a = g_ref[pl.ds(k * 16, 16)]
        g_b = g_ref[pl.ds(k * 16 + 1, 16)]
        a = jnp.where(sv_a == 0, 0.0, g_a)
        b = jnp.where(sv_b == 0, 0.0, g_b)
        node = lo + k * 16 + iot
        ob_ref[pl.ds(k * 16, 16)] = jnp.where(node < N_MOVABLE, b - a, 0.0)
        return carry

    lax.fori_loop(0, NODES_PER_W // 16, diff, 0)
    pltpu.sync_copy(ob_ref, out_hbm.at[pl.ds(lo, NODES_PER_W)])


def _build_bound():
    return pl.kernel(
        _bound_body,
        out_type=jax.ShapeDtypeStruct((OUT_PAD,), jnp.float32),
        mesh=plsc.VectorSubcoreMesh(
            core_axis_name="c", subcore_axis_name="s", num_cores=NC,
            num_subcores=NS,
        ),
        scratch_types=[
            pltpu.VMEM((SPAD,), jnp.int32),
            pltpu.VMEM((SPAD,), jnp.int32),
            pltpu.VMEM((SPAD,), jnp.float32),
            pltpu.VMEM((NODES_PER_W,), jnp.float32),
            pltpu.SemaphoreType.DMA,
        ],
    )


def kernel(net_weights, flat_node2pin_start, flat_node2pin, pin2net_map,
           flat_net2pin):
    w2d = jnp.pad(net_weights, (0, NET_PAD - N_NETS)).reshape(-1, 128)
    flo = jnp.pad(flat_net2pin[:N_NETS], (0, NET_PAD - N_NETS)).reshape(-1, 128)
    fhi = jnp.pad(flat_net2pin[1:], (0, NET_PAD - N_NETS)).reshape(-1, 128)
    v = _netval(w2d, flo, fhi).reshape(-1)
    val = _build_gather()(flat_node2pin, pin2net_map, v)
    s = _scan(val.reshape(ROWS, 128)).reshape(-1)
    stp = jnp.pad(flat_node2pin_start, (0, START_PAD - (N_NODES + 1)))
    return _build_bound()(s, stp)[:N_NODES]


# speedup vs baseline: 2086.3558x; 1.3131x over previous
import jax
import jax.numpy as jnp
from jax import lax
from jax.experimental import pallas as pl
from jax.experimental.pallas import tpu as pltpu
from jax.experimental.pallas import tpu_sc as plsc

N_NODES = 100000
N_MOVABLE = 90000
N_PINS = 3200000
N_NETS = 500000

NC, NS = 2, 16
NW = NC * NS
PIN_PER_W = N_PINS // NW
CHUNK = 2000
N_CHUNKS = PIN_PER_W // CHUNK

NET_PAD = 512000


def _netval_body(w_ref, flo_ref, fhi_ref, v_ref):
    d = fhi_ref[...] - flo_ref[...]
    w = jnp.maximum(w_ref[...], 1.0)
    den = jnp.maximum(d - 1, 1).astype(jnp.float32)
    v_ref[...] = jnp.where(d > 1, w / den, 0.0)


def _netval(w2d, flo2d, fhi2d):
    return pl.pallas_call(
        _netval_body,
        out_shape=jax.ShapeDtypeStruct((NET_PAD // 128, 128), jnp.float32),
    )(w2d, flo2d, fhi2d)


def _gather_body(n2p_hbm, p2n_hbm, v_hbm, out_hbm, idx_0, idx_1, net_0,
                 net_1, val_0, val_1, sem_a, sem_b, sem_c, sem_d):
    wid = lax.axis_index("c") * NS + lax.axis_index("s")
    pin0 = wid * PIN_PER_W
    idx_v = (idx_0, idx_1)
    net_v = (net_0, net_1)
    val_v = (val_0, val_1)

    def start_a(k, b):
        pltpu.async_copy(
            n2p_hbm.at[pl.ds(pin0 + k * CHUNK, CHUNK)], idx_v[b], sem_a)

    def wait_a(b):
        pltpu.make_async_copy(
            n2p_hbm.at[pl.ds(pin0, CHUNK)], idx_v[b], sem_a).wait()

    def start_b(b):
        pltpu.async_copy(p2n_hbm.at[idx_v[b]], net_v[b], sem_b)

    def wait_b(b):
        pltpu.make_async_copy(p2n_hbm.at[idx_v[b]], net_v[b], sem_b).wait()

    def start_c(b):
        pltpu.async_copy(v_hbm.at[net_v[b]], val_v[b], sem_c)

    def wait_c(b):
        pltpu.make_async_copy(v_hbm.at[net_v[b]], val_v[b], sem_c).wait()

    def start_d(k, b):
        pltpu.async_copy(
            val_v[b], out_hbm.at[pl.ds(pin0 + k * CHUNK, CHUNK)], sem_d)

    def wait_d(b):
        pltpu.make_async_copy(
            val_v[b], out_hbm.at[pl.ds(pin0, CHUNK)], sem_d).wait()

    start_a(0, 0)

    def step(k0, carry):
        for u in (0, 1):
            k = 2 * k0 + u

            @pl.when((k >= 2) & (k < N_CHUNKS + 2))
            def _(k=k, u=u):
                wait_c(u)
                start_d(k - 2, u)

            @pl.when((k >= 3) & (k < N_CHUNKS + 3))
            def _(u=u):
                wait_d(1 - u)

            @pl.when((k >= 1) & (k < N_CHUNKS + 1))
            def _(u=u):
                wait_b(1 - u)
                start_c(1 - u)

            @pl.when(k < N_CHUNKS)
            def _(k=k, u=u):
                wait_a(u)
                start_b(u)

            @pl.when(k + 1 < N_CHUNKS)
            def _(k=k, u=u):
                start_a(k + 1, 1 - u)

        return carry

    lax.fori_loop(0, (N_CHUNKS + 4) // 2, step, 0)


def _build_gather():
    return pl.kernel(
        _gather_body,
        out_type=jax.ShapeDtypeStruct((N_PINS,), jnp.float32),
        mesh=plsc.VectorSubcoreMesh(
            core_axis_name="c", subcore_axis_name="s", num_cores=NC,
            num_subcores=NS,
        ),
        scratch_types=[
            pltpu.VMEM((CHUNK,), jnp.int32),
            pltpu.VMEM((CHUNK,), jnp.int32),
            pltpu.VMEM((CHUNK,), jnp.int32),
            pltpu.VMEM((CHUNK,), jnp.int32),
            pltpu.VMEM((CHUNK,), jnp.float32),
            pltpu.VMEM((CHUNK,), jnp.float32),
            pltpu.SemaphoreType.DMA,
            pltpu.SemaphoreType.DMA,
            pltpu.SemaphoreType.DMA,
            pltpu.SemaphoreType.DMA,
        ],
    )


ROWS = N_PINS // 128
RBLK = 1000
NBLK = ROWS // RBLK


def _scan_body(x_ref, y_ref, carry_ref):
    @pl.when(pl.program_id(0) == 0)
    def _():
        carry_ref[0, 0] = 0.0

    x = x_ref[...]
    r = lax.broadcasted_iota(jnp.int32, (128, 128), 0)
    col = lax.broadcasted_iota(jnp.int32, (128, 128), 1)
    tri = (r <= col).astype(jnp.float32)
    y = jnp.dot(x, tri, preferred_element_type=jnp.float32)
    t = y[:, 127:128]
    e = jnp.concatenate([jnp.zeros((1, 1), jnp.float32), t[:-1, :]], axis=0)
    k = 1
    while k < RBLK:
        e = e + jnp.concatenate(
            [jnp.zeros((k, 1), jnp.float32), e[:-k, :]], axis=0
        )
        k *= 2
    c = carry_ref[0, 0]
    y_ref[...] = y + e + c
    carry_ref[0, 0] = c + jnp.sum(t[RBLK - 1:, :]) + jnp.sum(e[RBLK - 1:, :])


def _scan(val2d):
    return pl.pallas_call(
        _scan_body,
        grid=(NBLK,),
        in_specs=[pl.BlockSpec((RBLK, 128), lambda i: (i, 0))],
        out_specs=pl.BlockSpec((RBLK, 128), lambda i: (i, 0)),
        out_shape=jax.ShapeDtypeStruct((ROWS, 128), jnp.float32),
        scratch_shapes=[pltpu.SMEM((1, 1), jnp.float32)],
    )(val2d)


NODES_PER_W = 3136
OUT_PAD = NW * NODES_PER_W
SPAD = NODES_PER_W + 16
START_PAD = (NW - 1) * NODES_PER_W + SPAD


def _bound_body(s_hbm, st_hbm, out_hbm, sv_ref, idx_ref, g_ref, ob_ref, sem):
    wid = lax.axis_index("c") * NS + lax.axis_index("s")
    lo = wid * NODES_PER_W
    pltpu.sync_copy(st_hbm.at[pl.ds(lo, SPAD)], sv_ref)

    def mk_idx(k, carry):
        sv = sv_ref[pl.ds(k * 16, 16)]
        idx_ref[pl.ds(k * 16, 16)] = jnp.maximum(sv - 1, 0)
        return carry

    lax.fori_loop(0, SPAD // 16, mk_idx, 0)
    pltpu.async_copy(s_hbm.at[idx_ref], g_ref, sem).wait()
    iot = lax.iota(jnp.int32, 16)

    def diff(k, carry):
        sv_a = sv_ref[pl.ds(k * 16, 16)]
        sv_b = sv_ref[pl.ds(k * 16 + 1, 16)]
        g_a = g_ref[pl.ds(k * 16, 16)]
        g_b = g_ref[pl.ds(k * 16 + 1, 16)]
        a = jnp.where(sv_a == 0, 0.0, g_a)
        b = jnp.where(sv_b == 0, 0.0, g_b)
        node = lo + k * 16 + iot
        ob_ref[pl.ds(k * 16, 16)] = jnp.where(node < N_MOVABLE, b - a, 0.0)
        return carry

    lax.fori_loop(0, NODES_PER_W // 16, diff, 0)
    pltpu.sync_copy(ob_ref, out_hbm.at[pl.ds(lo, NODES_PER_W)])


def _build_bound():
    return pl.kernel(
        _bound_body,
        out_type=jax.ShapeDtypeStruct((OUT_PAD,), jnp.float32),
        mesh=plsc.VectorSubcoreMesh(
            core_axis_name="c", subcore_axis_name="s", num_cores=NC,
            num_subcores=NS,
        ),
        scratch_types=[
            pltpu.VMEM((SPAD,), jnp.int32),
            pltpu.VMEM((SPAD,), jnp.int32),
            pltpu.VMEM((SPAD,), jnp.float32),
            pltpu.VMEM((NODES_PER_W,), jnp.float32),
            pltpu.SemaphoreType.DMA,
        ],
    )


def kernel(net_weights, flat_node2pin_start, flat_node2pin, pin2net_map,
           flat_net2pin):
    w2d = jnp.pad(net_weights, (0, NET_PAD - N_NETS)).reshape(-1, 128)
    flo = jnp.pad(flat_net2pin[:N_NETS], (0, NET_PAD - N_NETS)).reshape(-1, 128)
    fhi = jnp.pad(flat_net2pin[1:], (0, NET_PAD - N_NETS)).reshape(-1, 128)
    v = _netval(w2d, flo, fhi).reshape(-1)
    val = _build_gather()(flat_node2pin, pin2net_map, v)
    s = _scan(val.reshape(ROWS, 128)).reshape(-1)
    stp = jnp.pad(flat_node2pin_start, (0, START_PAD - (N_NODES + 1)))
    return _build_bound()(s, stp)[:N_NODES]


# speedup vs baseline: 2108.5665x; 1.0106x over previous
import jax
import jax.numpy as jnp
from jax import lax
from jax.experimental import pallas as pl
from jax.experimental.pallas import tpu as pltpu
from jax.experimental.pallas import tpu_sc as plsc

N_NODES = 100000
N_MOVABLE = 90000
N_PINS = 3200000
N_NETS = 500000

NC, NS = 2, 16
NW = NC * NS
PIN_PER_W = N_PINS // NW
CHUNK = 4000
N_CHUNKS = PIN_PER_W // CHUNK

NET_PAD = 512000


def _netval_body(w_ref, flo_ref, fhi_ref, v_ref):
    d = fhi_ref[...] - flo_ref[...]
    w = jnp.maximum(w_ref[...], 1.0)
    den = jnp.maximum(d - 1, 1).astype(jnp.float32)
    v_ref[...] = jnp.where(d > 1, w / den, 0.0)


def _netval(w2d, flo2d, fhi2d):
    return pl.pallas_call(
        _netval_body,
        out_shape=jax.ShapeDtypeStruct((NET_PAD // 128, 128), jnp.float32),
    )(w2d, flo2d, fhi2d)


def _gather_body(n2p_hbm, p2n_hbm, v_hbm, out_hbm, idx_0, idx_1, net_0,
                 net_1, val_0, val_1, sem_a, sem_b, sem_c, sem_d):
    wid = lax.axis_index("c") * NS + lax.axis_index("s")
    pin0 = wid * PIN_PER_W
    idx_v = (idx_0, idx_1)
    net_v = (net_0, net_1)
    val_v = (val_0, val_1)

    def start_a(k, b):
        pltpu.async_copy(
            n2p_hbm.at[pl.ds(pin0 + k * CHUNK, CHUNK)], idx_v[b], sem_a)

    def wait_a(b):
        pltpu.make_async_copy(
            n2p_hbm.at[pl.ds(pin0, CHUNK)], idx_v[b], sem_a).wait()

    def start_b(b):
        pltpu.async_copy(p2n_hbm.at[idx_v[b]], net_v[b], sem_b)

    def wait_b(b):
        pltpu.make_async_copy(p2n_hbm.at[idx_v[b]], net_v[b], sem_b).wait()

    def start_c(b):
        pltpu.async_copy(v_hbm.at[net_v[b]], val_v[b], sem_c)

    def wait_c(b):
        pltpu.make_async_copy(v_hbm.at[net_v[b]], val_v[b], sem_c).wait()

    def start_d(k, b):
        pltpu.async_copy(
            val_v[b], out_hbm.at[pl.ds(pin0 + k * CHUNK, CHUNK)], sem_d)

    def wait_d(b):
        pltpu.make_async_copy(
            val_v[b], out_hbm.at[pl.ds(pin0, CHUNK)], sem_d).wait()

    start_a(0, 0)

    def step(k0, carry):
        for u in (0, 1):
            k = 2 * k0 + u

            @pl.when((k >= 2) & (k < N_CHUNKS + 2))
            def _(k=k, u=u):
                wait_c(u)
                start_d(k - 2, u)

            @pl.when((k >= 3) & (k < N_CHUNKS + 3))
            def _(u=u):
                wait_d(1 - u)

            @pl.when((k >= 1) & (k < N_CHUNKS + 1))
            def _(u=u):
                wait_b(1 - u)
                start_c(1 - u)

            @pl.when(k < N_CHUNKS)
            def _(k=k, u=u):
                wait_a(u)
                start_b(u)

            @pl.when(k + 1 < N_CHUNKS)
            def _(k=k, u=u):
                start_a(k + 1, 1 - u)

        return carry

    lax.fori_loop(0, (N_CHUNKS + 4) // 2, step, 0)


def _build_gather():
    return pl.kernel(
        _gather_body,
        out_type=jax.ShapeDtypeStruct((N_PINS,), jnp.float32),
        mesh=plsc.VectorSubcoreMesh(
            core_axis_name="c", subcore_axis_name="s", num_cores=NC,
            num_subcores=NS,
        ),
        scratch_types=[
            pltpu.VMEM((CHUNK,), jnp.int32),
            pltpu.VMEM((CHUNK,), jnp.int32),
            pltpu.VMEM((CHUNK,), jnp.int32),
            pltpu.VMEM((CHUNK,), jnp.int32),
            pltpu.VMEM((CHUNK,), jnp.float32),
            pltpu.VMEM((CHUNK,), jnp.float32),
            pltpu.SemaphoreType.DMA,
            pltpu.SemaphoreType.DMA,
            pltpu.SemaphoreType.DMA,
            pltpu.SemaphoreType.DMA,
        ],
    )


ROWS = N_PINS // 128
RBLK = 1000
NBLK = ROWS // RBLK


def _scan_body(x_ref, y_ref, carry_ref):
    @pl.when(pl.program_id(0) == 0)
    def _():
        carry_ref[0, 0] = 0.0

    x = x_ref[...]
    r = lax.broadcasted_iota(jnp.int32, (128, 128), 0)
    col = lax.broadcasted_iota(jnp.int32, (128, 128), 1)
    tri = (r <= col).astype(jnp.float32)
    y = jnp.dot(x, tri, preferred_element_type=jnp.float32)
    t = y[:, 127:128]
    e = jnp.concatenate([jnp.zeros((1, 1), jnp.float32), t[:-1, :]], axis=0)
    k = 1
    while k < RBLK:
        e = e + jnp.concatenate(
            [jnp.zeros((k, 1), jnp.float32), e[:-k, :]], axis=0
        )
        k *= 2
    c = carry_ref[0, 0]
    y_ref[...] = y + e + c
    carry_ref[0, 0] = c + jnp.sum(t[RBLK - 1:, :]) + jnp.sum(e[RBLK - 1:, :])


def _scan(val2d):
    return pl.pallas_call(
        _scan_body,
        grid=(NBLK,),
        in_specs=[pl.BlockSpec((RBLK, 128), lambda i: (i, 0))],
        out_specs=pl.BlockSpec((RBLK, 128), lambda i: (i, 0)),
        out_shape=jax.ShapeDtypeStruct((ROWS, 128), jnp.float32),
        scratch_shapes=[pltpu.SMEM((1, 1), jnp.float32)],
    )(val2d)


NODES_PER_W = 3136
OUT_PAD = NW * NODES_PER_W
SPAD = NODES_PER_W + 16
START_PAD = (NW - 1) * NODES_PER_W + SPAD


def _bound_body(s_hbm, st_hbm, out_hbm, sv_ref, idx_ref, g_ref, ob_ref, sem):
    wid = lax.axis_index("c") * NS + lax.axis_index("s")
    lo = wid * NODES_PER_W
    pltpu.sync_copy(st_hbm.at[pl.ds(lo, SPAD)], sv_ref)

    def mk_idx(k, carry):
        sv = sv_ref[pl.ds(k * 16, 16)]
        idx_ref[pl.ds(k * 16, 16)] = jnp.maximum(sv - 1, 0)
        return carry

    lax.fori_loop(0, SPAD // 16, mk_idx, 0)
    pltpu.async_copy(s_hbm.at[idx_ref], g_ref, sem).wait()
    iot = lax.iota(jnp.int32, 16)

    def diff(k, carry):
        sv_a = sv_ref[pl.ds(k * 16, 16)]
        sv_b = sv_ref[pl.ds(k * 16 + 1, 16)]
        g_a = g_ref[pl.ds(k * 16, 16)]
        g_b = g_ref[pl.ds(k * 16 + 1, 16)]
        a = jnp.where(sv_a == 0, 0.0, g_a)
        b = jnp.where(sv_b == 0, 0.0, g_b)
        node = lo + k * 16 + iot
        ob_ref[pl.ds(k * 16, 16)] = jnp.where(node < N_MOVABLE, b - a, 0.0)
        return carry

    lax.fori_loop(0, NODES_PER_W // 16, diff, 0)
    pltpu.sync_copy(ob_ref, out_hbm.at[pl.ds(lo, NODES_PER_W)])


def _build_bound():
    return pl.kernel(
        _bound_body,
        out_type=jax.ShapeDtypeStruct((OUT_PAD,), jnp.float32),
        mesh=plsc.VectorSubcoreMesh(
            core_axis_name="c", subcore_axis_name="s", num_cores=NC,
            num_subcores=NS,
        ),
        scratch_types=[
            pltpu.VMEM((SPAD,), jnp.int32),
            pltpu.VMEM((SPAD,), jnp.int32),
            pltpu.VMEM((SPAD,), jnp.float32),
            pltpu.VMEM((NODES_PER_W,), jnp.float32),
            pltpu.SemaphoreType.DMA,
        ],
    )


def kernel(net_weights, flat_node2pin_start, flat_node2pin, pin2net_map,
           flat_net2pin):
    w2d = jnp.pad(net_weights, (0, NET_PAD - N_NETS)).reshape(-1, 128)
    flo = jnp.pad(flat_net2pin[:N_NETS], (0, NET_PAD - N_NETS)).reshape(-1, 128)
    fhi = jnp.pad(flat_net2pin[1:], (0, NET_PAD - N_NETS)).reshape(-1, 128)
    v = _netval(w2d, flo, fhi).reshape(-1)
    val = _build_gather()(flat_node2pin, pin2net_map, v)
    s = _scan(val.reshape(ROWS, 128)).reshape(-1)
    stp = jnp.pad(flat_node2pin_start, (0, START_PAD - (N_NODES + 1)))
    return _build_bound()(s, stp)[:N_NODES]


# speedup vs baseline: 3197.6266x; 1.5165x over previous
import jax
import jax.numpy as jnp
from jax import lax
from jax.experimental import pallas as pl
from jax.experimental.pallas import tpu as pltpu
from jax.experimental.pallas import tpu_sc as plsc

N_NODES = 100000
N_MOVABLE = 90000
N_PINS = 3200000
N_NETS = 500000

NC, NS = 2, 16
NW = NC * NS
PIN_PER_W = N_PINS // NW
CHUNK = 4000
N_CHUNKS = PIN_PER_W // CHUNK

NET_PAD = 512000


def _netval_body(w_ref, flo_ref, fhi_ref, v_ref):
    d = fhi_ref[...] - flo_ref[...]
    w = jnp.maximum(w_ref[...], 1.0)
    den = jnp.maximum(d - 1, 1).astype(jnp.float32)
    v_ref[...] = jnp.where(d > 1, w / den, 0.0)


def _netval(w2d, flo2d, fhi2d):
    return pl.pallas_call(
        _netval_body,
        out_shape=jax.ShapeDtypeStruct((NET_PAD // 128, 128), jnp.float32),
    )(w2d, flo2d, fhi2d)


def _gather_body(n2p_hbm, p2n_hbm, v_hbm, out_hbm, idx_0, idx_1, net_0,
                 net_1, val_0, val_1, v_sp, sem_a, sem_b, sem_c, sem_d):
    sid = lax.axis_index("s")
    wid = lax.axis_index("c") * NS + sid
    pin0 = wid * PIN_PER_W
    idx_v = (idx_0, idx_1)
    net_v = (net_0, net_1)
    val_v = (val_0, val_1)

    vseg = NET_PAD // NS
    pltpu.sync_copy(v_hbm.at[pl.ds(sid * vseg, vseg)],
                    v_sp.at[pl.ds(sid * vseg, vseg)])
    plsc.subcore_barrier()

    def start_a(k, b):
        pltpu.async_copy(
            n2p_hbm.at[pl.ds(pin0 + k * CHUNK, CHUNK)], idx_v[b], sem_a)

    def wait_a(b):
        pltpu.make_async_copy(
            n2p_hbm.at[pl.ds(pin0, CHUNK)], idx_v[b], sem_a).wait()

    def start_b(b):
        pltpu.async_copy(p2n_hbm.at[idx_v[b]], net_v[b], sem_b)

    def wait_b(b):
        pltpu.make_async_copy(p2n_hbm.at[idx_v[b]], net_v[b], sem_b).wait()

    def start_c(b):
        pltpu.async_copy(v_sp.at[net_v[b]], val_v[b], sem_c)

    def wait_c(b):
        pltpu.make_async_copy(v_sp.at[net_v[b]], val_v[b], sem_c).wait()

    def start_d(k, b):
        pltpu.async_copy(
            val_v[b], out_hbm.at[pl.ds(pin0 + k * CHUNK, CHUNK)], sem_d)

    def wait_d(b):
        pltpu.make_async_copy(
            val_v[b], out_hbm.at[pl.ds(pin0, CHUNK)], sem_d).wait()

    start_a(0, 0)

    def step(k0, carry):
        for u in (0, 1):
            k = 2 * k0 + u

            @pl.when((k >= 2) & (k < N_CHUNKS + 2))
            def _(k=k, u=u):
                wait_c(u)
                start_d(k - 2, u)

            @pl.when((k >= 3) & (k < N_CHUNKS + 3))
            def _(u=u):
                wait_d(1 - u)

            @pl.when((k >= 1) & (k < N_CHUNKS + 1))
            def _(u=u):
                wait_b(1 - u)
                start_c(1 - u)

            @pl.when(k < N_CHUNKS)
            def _(k=k, u=u):
                wait_a(u)
                start_b(u)

            @pl.when(k + 1 < N_CHUNKS)
            def _(k=k, u=u):
                start_a(k + 1, 1 - u)

        return carry

    lax.fori_loop(0, (N_CHUNKS + 4) // 2, step, 0)


def _build_gather():
    return pl.kernel(
        _gather_body,
        out_type=jax.ShapeDtypeStruct((N_PINS,), jnp.float32),
        mesh=plsc.VectorSubcoreMesh(
            core_axis_name="c", subcore_axis_name="s", num_cores=NC,
            num_subcores=NS,
        ),
        scratch_types=[
            pltpu.VMEM((CHUNK,), jnp.int32),
            pltpu.VMEM((CHUNK,), jnp.int32),
            pltpu.VMEM((CHUNK,), jnp.int32),
            pltpu.VMEM((CHUNK,), jnp.int32),
            pltpu.VMEM((CHUNK,), jnp.float32),
            pltpu.VMEM((CHUNK,), jnp.float32),
            pltpu.VMEM_SHARED((NET_PAD,), jnp.float32),
            pltpu.SemaphoreType.DMA,
            pltpu.SemaphoreType.DMA,
            pltpu.SemaphoreType.DMA,
            pltpu.SemaphoreType.DMA,
        ],
    )


ROWS = N_PINS // 128
RBLK = 1000
NBLK = ROWS // RBLK


def _scan_body(x_ref, y_ref, carry_ref):
    @pl.when(pl.program_id(0) == 0)
    def _():
        carry_ref[0, 0] = 0.0

    x = x_ref[...]
    r = lax.broadcasted_iota(jnp.int32, (128, 128), 0)
    col = lax.broadcasted_iota(jnp.int32, (128, 128), 1)
    tri = (r <= col).astype(jnp.float32)
    y = jnp.dot(x, tri, preferred_element_type=jnp.float32)
    t = y[:, 127:128]
    e = jnp.concatenate([jnp.zeros((1, 1), jnp.float32), t[:-1, :]], axis=0)
    k = 1
    while k < RBLK:
        e = e + jnp.concatenate(
            [jnp.zeros((k, 1), jnp.float32), e[:-k, :]], axis=0
        )
        k *= 2
    c = carry_ref[0, 0]
    y_ref[...] = y + e + c
    carry_ref[0, 0] = c + jnp.sum(t[RBLK - 1:, :]) + jnp.sum(e[RBLK - 1:, :])


def _scan(val2d):
    return pl.pallas_call(
        _scan_body,
        grid=(NBLK,),
        in_specs=[pl.BlockSpec((RBLK, 128), lambda i: (i, 0))],
        out_specs=pl.BlockSpec((RBLK, 128), lambda i: (i, 0)),
        out_shape=jax.ShapeDtypeStruct((ROWS, 128), jnp.float32),
        scratch_shapes=[pltpu.SMEM((1, 1), jnp.float32)],
    )(val2d)


NODES_PER_W = 3136
OUT_PAD = NW * NODES_PER_W
SPAD = NODES_PER_W + 16
START_PAD = (NW - 1) * NODES_PER_W + SPAD


def _bound_body(s_hbm, st_hbm, out_hbm, sv_ref, idx_ref, g_ref, ob_ref, sem):
    wid = lax.axis_index("c") * NS + lax.axis_index("s")
    lo = wid * NODES_PER_W
    pltpu.sync_copy(st_hbm.at[pl.ds(lo, SPAD)], sv_ref)

    def mk_idx(k, carry):
        sv = sv_ref[pl.ds(k * 16, 16)]
        idx_ref[pl.ds(k * 16, 16)] = jnp.maximum(sv - 1, 0)
        return carry

    lax.fori_loop(0, SPAD // 16, mk_idx, 0)
    pltpu.async_copy(s_hbm.at[idx_ref], g_ref, sem).wait()
    iot = lax.iota(jnp.int32, 16)

    def diff(k, carry):
        sv_a = sv_ref[pl.ds(k * 16, 16)]
        sv_b = sv_ref[pl.ds(k * 16 + 1, 16)]
        g_a = g_ref[pl.ds(k * 16, 16)]
        g_b = g_ref[pl.ds(k * 16 + 1, 16)]
        a = jnp.where(sv_a == 0, 0.0, g_a)
        b = jnp.where(sv_b == 0, 0.0, g_b)
        node = lo + k * 16 + iot
        ob_ref[pl.ds(k * 16, 16)] = jnp.where(node < N_MOVABLE, b - a, 0.0)
        return carry

    lax.fori_loop(0, NODES_PER_W // 16, diff, 0)
    pltpu.sync_copy(ob_ref, out_hbm.at[pl.ds(lo, NODES_PER_W)])


def _build_bound():
    return pl.kernel(
        _bound_body,
        out_type=jax.ShapeDtypeStruct((OUT_PAD,), jnp.float32),
        mesh=plsc.VectorSubcoreMesh(
            core_axis_name="c", subcore_axis_name="s", num_cores=NC,
            num_subcores=NS,
        ),
        scratch_types=[
            pltpu.VMEM((SPAD,), jnp.int32),
            pltpu.VMEM((SPAD,), jnp.int32),
            pltpu.VMEM((SPAD,), jnp.float32),
            pltpu.VMEM((NODES_PER_W,), jnp.float32),
            pltpu.SemaphoreType.DMA,
        ],
    )


def kernel(net_weights, flat_node2pin_start, flat_node2pin, pin2net_map,
           flat_net2pin):
    w2d = jnp.pad(net_weights, (0, NET_PAD - N_NETS)).reshape(-1, 128)
    flo = jnp.pad(flat_net2pin[:N_NETS], (0, NET_PAD - N_NETS)).reshape(-1, 128)
    fhi = jnp.pad(flat_net2pin[1:], (0, NET_PAD - N_NETS)).reshape(-1, 128)
    v = _netval(w2d, flo, fhi).reshape(-1)
    val = _build_gather()(flat_node2pin, pin2net_map, v)
    s = _scan(val.reshape(ROWS, 128)).reshape(-1)
    stp = jnp.pad(flat_node2pin_start, (0, START_PAD - (N_NODES + 1)))
    return _build_bound()(s, stp)[:N_NODES]


# speedup vs baseline: 3237.2016x; 1.0124x over previous
import jax
import jax.numpy as jnp
from jax import lax
from jax.experimental import pallas as pl
from jax.experimental.pallas import tpu as pltpu
from jax.experimental.pallas import tpu_sc as plsc

N_NODES = 100000
N_MOVABLE = 90000
N_PINS = 3200000
N_NETS = 500000

NC, NS = 2, 16
NW = NC * NS
PIN_PER_W = N_PINS // NW
CHUNK = 4000
N_CHUNKS = PIN_PER_W // CHUNK

NET_PAD = 512000
F2P_PAD = 512016
NSEG = NET_PAD // NS
SUB = 8000


def _gather_body(n2p_hbm, p2n_hbm, w_hbm, f2p_hbm, out_hbm, idx_0, idx_1,
                 net_0, net_1, val_0, val_1, w_buf, f_buf, v_buf, v_sp,
                 sem_a, sem_b, sem_c, sem_d):
    sid = lax.axis_index("s")
    wid = lax.axis_index("c") * NS + sid
    pin0 = wid * PIN_PER_W
    idx_v = (idx_0, idx_1)
    net_v = (net_0, net_1)
    val_v = (val_0, val_1)

    for t in range(NSEG // SUB):
        g0 = sid * NSEG + t * SUB
        pltpu.sync_copy(w_hbm.at[pl.ds(g0, SUB)], w_buf)
        pltpu.sync_copy(f2p_hbm.at[pl.ds(g0, SUB + 16)], f_buf)

        def vec(i, carry):
            flo = f_buf[pl.ds(i * 16, 16)]
            fhi = f_buf[pl.ds(i * 16 + 1, 16)]
            d = fhi - flo
            w16 = jnp.maximum(w_buf[pl.ds(i * 16, 16)], 1.0)
            den = jnp.maximum(d - 1, 1).astype(jnp.float32)
            v_buf[pl.ds(i * 16, 16)] = jnp.where(d > 1, w16 / den, 0.0)
            return carry

        lax.fori_loop(0, SUB // 16, vec, 0)
        pltpu.sync_copy(v_buf, v_sp.at[pl.ds(g0, SUB)])
    plsc.subcore_barrier()

    def start_a(k, b):
        pltpu.async_copy(
            n2p_hbm.at[pl.ds(pin0 + k * CHUNK, CHUNK)], idx_v[b], sem_a)

    def wait_a(b):
        pltpu.make_async_copy(
            n2p_hbm.at[pl.ds(pin0, CHUNK)], idx_v[b], sem_a).wait()

    def start_b(b):
        pltpu.async_copy(p2n_hbm.at[idx_v[b]], net_v[b], sem_b)

    def wait_b(b):
        pltpu.make_async_copy(p2n_hbm.at[idx_v[b]], net_v[b], sem_b).wait()

    def start_c(b):
        pltpu.async_copy(v_sp.at[net_v[b]], val_v[b], sem_c)

    def wait_c(b):
        pltpu.make_async_copy(v_sp.at[net_v[b]], val_v[b], sem_c).wait()

    def start_d(k, b):
        pltpu.async_copy(
            val_v[b], out_hbm.at[pl.ds(pin0 + k * CHUNK, CHUNK)], sem_d)

    def wait_d(b):
        pltpu.make_async_copy(
            val_v[b], out_hbm.at[pl.ds(pin0, CHUNK)], sem_d).wait()

    start_a(0, 0)

    def step(k0, carry):
        for u in (0, 1):
            k = 2 * k0 + u

            @pl.when((k >= 2) & (k < N_CHUNKS + 2))
            def _(k=k, u=u):
                wait_c(u)
                start_d(k - 2, u)

            @pl.when((k >= 3) & (k < N_CHUNKS + 3))
            def _(u=u):
                wait_d(1 - u)

            @pl.when((k >= 1) & (k < N_CHUNKS + 1))
            def _(u=u):
                wait_b(1 - u)
                start_c(1 - u)

            @pl.when(k < N_CHUNKS)
            def _(k=k, u=u):
                wait_a(u)
                start_b(u)

            @pl.when(k + 1 < N_CHUNKS)
            def _(k=k, u=u):
                start_a(k + 1, 1 - u)

        return carry

    lax.fori_loop(0, (N_CHUNKS + 4) // 2, step, 0)


def _build_gather():
    return pl.kernel(
        _gather_body,
        out_type=jax.ShapeDtypeStruct((N_PINS,), jnp.float32),
        mesh=plsc.VectorSubcoreMesh(
            core_axis_name="c", subcore_axis_name="s", num_cores=NC,
            num_subcores=NS,
        ),
        scratch_types=[
            pltpu.VMEM((CHUNK,), jnp.int32),
            pltpu.VMEM((CHUNK,), jnp.int32),
            pltpu.VMEM((CHUNK,), jnp.int32),
            pltpu.VMEM((CHUNK,), jnp.int32),
            pltpu.VMEM((CHUNK,), jnp.float32),
            pltpu.VMEM((CHUNK,), jnp.float32),
            pltpu.VMEM((SUB,), jnp.float32),
            pltpu.VMEM((SUB + 16,), jnp.int32),
            pltpu.VMEM((SUB,), jnp.float32),
            pltpu.VMEM_SHARED((NET_PAD,), jnp.float32),
            pltpu.SemaphoreType.DMA,
            pltpu.SemaphoreType.DMA,
            pltpu.SemaphoreType.DMA,
            pltpu.SemaphoreType.DMA,
        ],
    )


ROWS = N_PINS // 128
RBLK = 5000
NBLK = ROWS // RBLK


def _scan_body(x_ref, y_ref, carry_ref):
    @pl.when(pl.program_id(0) == 0)
    def _():
        carry_ref[0, 0] = 0.0

    x = x_ref[...]
    r = lax.broadcasted_iota(jnp.int32, (128, 128), 0)
    col = lax.broadcasted_iota(jnp.int32, (128, 128), 1)
    tri = (r <= col).astype(jnp.float32)
    y = jnp.dot(x, tri, preferred_element_type=jnp.float32)
    t = y[:, 127:128]
    e = jnp.concatenate([jnp.zeros((1, 1), jnp.float32), t[:-1, :]], axis=0)
    k = 1
    while k < RBLK:
        e = e + jnp.concatenate(
            [jnp.zeros((k, 1), jnp.float32), e[:-k, :]], axis=0
        )
        k *= 2
    c = carry_ref[0, 0]
    y_ref[...] = y + e + c
    carry_ref[0, 0] = c + jnp.sum(t[RBLK - 1:, :]) + jnp.sum(e[RBLK - 1:, :])


def _scan(val2d):
    return pl.pallas_call(
        _scan_body,
        grid=(NBLK,),
        in_specs=[pl.BlockSpec((RBLK, 128), lambda i: (i, 0))],
        out_specs=pl.BlockSpec((RBLK, 128), lambda i: (i, 0)),
        out_shape=jax.ShapeDtypeStruct((ROWS, 128), jnp.float32),
        scratch_shapes=[pltpu.SMEM((1, 1), jnp.float32)],
    )(val2d)


NODES_PER_W = 3136
OUT_PAD = NW * NODES_PER_W
SPAD = NODES_PER_W + 16
START_PAD = (NW - 1) * NODES_PER_W + SPAD


def _bound_body(s_hbm, st_hbm, out_hbm, sv_ref, idx_ref, g_ref, ob_ref, sem):
    wid = lax.axis_index("c") * NS + lax.axis_index("s")
    lo = wid * NODES_PER_W
    pltpu.sync_copy(st_hbm.at[pl.ds(lo, SPAD)], sv_ref)

    def mk_idx(k, carry):
        sv = sv_ref[pl.ds(k * 16, 16)]
        idx_ref[pl.ds(k * 16, 16)] = jnp.maximum(sv - 1, 0)
        return carry

    lax.fori_loop(0, SPAD // 16, mk_idx, 0)
    pltpu.async_copy(s_hbm.at[idx_ref], g_ref, sem).wait()
    iot = lax.iota(jnp.int32, 16)

    def diff(k, carry):
        sv_a = sv_ref[pl.ds(k * 16, 16)]
        sv_b = sv_ref[pl.ds(k * 16 + 1, 16)]
        g_a = g_ref[pl.ds(k * 16, 16)]
        g_b = g_ref[pl.ds(k * 16 + 1, 16)]
        a = jnp.where(sv_a == 0, 0.0, g_a)
        b = jnp.where(sv_b == 0, 0.0, g_b)
        node = lo + k * 16 + iot
        ob_ref[pl.ds(k * 16, 16)] = jnp.where(node < N_MOVABLE, b - a, 0.0)
        return carry

    lax.fori_loop(0, NODES_PER_W // 16, diff, 0)
    pltpu.sync_copy(ob_ref, out_hbm.at[pl.ds(lo, NODES_PER_W)])


def _build_bound():
    return pl.kernel(
        _bound_body,
        out_type=jax.ShapeDtypeStruct((OUT_PAD,), jnp.float32),
        mesh=plsc.VectorSubcoreMesh(
            core_axis_name="c", subcore_axis_name="s", num_cores=NC,
            num_subcores=NS,
        ),
        scratch_types=[
            pltpu.VMEM((SPAD,), jnp.int32),
            pltpu.VMEM((SPAD,), jnp.int32),
            pltpu.VMEM((SPAD,), jnp.float32),
            pltpu.VMEM((NODES_PER_W,), jnp.float32),
            pltpu.SemaphoreType.DMA,
        ],
    )


def kernel(net_weights, flat_node2pin_start, flat_node2pin, pin2net_map,
           flat_net2pin):
    w_pad = jnp.pad(net_weights, (0, NET_PAD - N_NETS))
    f2p_pad = jnp.pad(flat_net2pin, (0, F2P_PAD - (N_NETS + 1)))
    val = _build_gather()(flat_node2pin, pin2net_map, w_pad, f2p_pad)
    s = _scan(val.reshape(ROWS, 128)).reshape(-1)
    stp = jnp.pad(flat_node2pin_start, (0, START_PAD - (N_NODES + 1)))
    return _build_bound()(s, stp)[:N_NODES]


# speedup vs baseline: 3427.1520x; 1.0587x over previous
import jax
import jax.numpy as jnp
from jax import lax
from jax.experimental import pallas as pl
from jax.experimental.pallas import tpu as pltpu
from jax.experimental.pallas import tpu_sc as plsc

N_NODES = 100000
N_MOVABLE = 90000
N_PINS = 3200000
N_NETS = 500000

NC, NS = 2, 16
NW = NC * NS
PIN_PER_W = N_PINS // NW
CHUNK = 4000
N_CHUNKS = PIN_PER_W // CHUNK

NET_PAD = 512000
F2P_PAD = 512016
NSEG = NET_PAD // NS
SUB = 8000


def _gather_body(n2p_hbm, p2n_hbm, w_hbm, f2p_hbm, out_hbm, idx_0, idx_1,
                 net_0, net_1, val_0, val_1, w_buf, f_buf, v_buf, v_sp,
                 sem_a, sem_b, sem_b2, sem_c, sem_d):
    sid = lax.axis_index("s")
    wid = lax.axis_index("c") * NS + sid
    pin0 = wid * PIN_PER_W
    idx_v = (idx_0, idx_1)
    net_v = (net_0, net_1)
    val_v = (val_0, val_1)

    for t in range(NSEG // SUB):
        g0 = sid * NSEG + t * SUB
        pltpu.sync_copy(w_hbm.at[pl.ds(g0, SUB)], w_buf)
        pltpu.sync_copy(f2p_hbm.at[pl.ds(g0, SUB + 16)], f_buf)

        def vec(i, carry):
            flo = f_buf[pl.ds(i * 16, 16)]
            fhi = f_buf[pl.ds(i * 16 + 1, 16)]
            d = fhi - flo
            w16 = jnp.maximum(w_buf[pl.ds(i * 16, 16)], 1.0)
            den = jnp.maximum(d - 1, 1).astype(jnp.float32)
            v_buf[pl.ds(i * 16, 16)] = jnp.where(d > 1, w16 / den, 0.0)
            return carry

        lax.fori_loop(0, SUB // 16, vec, 0)
        pltpu.sync_copy(v_buf, v_sp.at[pl.ds(g0, SUB)])
    plsc.subcore_barrier()

    def start_a(k, b):
        pltpu.async_copy(
            n2p_hbm.at[pl.ds(pin0 + k * CHUNK, CHUNK)], idx_v[b], sem_a)

    def wait_a(b):
        pltpu.make_async_copy(
            n2p_hbm.at[pl.ds(pin0, CHUNK)], idx_v[b], sem_a).wait()

    half = CHUNK // 2

    def start_b(b):
        pltpu.async_copy(p2n_hbm.at[idx_v[b].at[pl.ds(0, half)]],
                         net_v[b].at[pl.ds(0, half)], sem_b)
        pltpu.async_copy(p2n_hbm.at[idx_v[b].at[pl.ds(half, half)]],
                         net_v[b].at[pl.ds(half, half)], sem_b2)

    def wait_b(b):
        pltpu.make_async_copy(p2n_hbm.at[idx_v[b].at[pl.ds(0, half)]],
                              net_v[b].at[pl.ds(0, half)], sem_b).wait()
        pltpu.make_async_copy(p2n_hbm.at[idx_v[b].at[pl.ds(half, half)]],
                              net_v[b].at[pl.ds(half, half)], sem_b2).wait()

    def start_c(b):
        pltpu.async_copy(v_sp.at[net_v[b]], val_v[b], sem_c)

    def wait_c(b):
        pltpu.make_async_copy(v_sp.at[net_v[b]], val_v[b], sem_c).wait()

    def start_d(k, b):
        pltpu.async_copy(
            val_v[b], out_hbm.at[pl.ds(pin0 + k * CHUNK, CHUNK)], sem_d)

    def wait_d(b):
        pltpu.make_async_copy(
            val_v[b], out_hbm.at[pl.ds(pin0, CHUNK)], sem_d).wait()

    start_a(0, 0)

    def step(k0, carry):
        for u in (0, 1):
            k = 2 * k0 + u

            @pl.when((k >= 2) & (k < N_CHUNKS + 2))
            def _(k=k, u=u):
                wait_c(u)
                start_d(k - 2, u)

            @pl.when((k >= 3) & (k < N_CHUNKS + 3))
            def _(u=u):
                wait_d(1 - u)

            @pl.when((k >= 1) & (k < N_CHUNKS + 1))
            def _(u=u):
                wait_b(1 - u)
                start_c(1 - u)

            @pl.when(k < N_CHUNKS)
            def _(k=k, u=u):
                wait_a(u)
                start_b(u)

            @pl.when(k + 1 < N_CHUNKS)
            def _(k=k, u=u):
                start_a(k + 1, 1 - u)

        return carry

    lax.fori_loop(0, (N_CHUNKS + 4) // 2, step, 0)


def _build_gather():
    return pl.kernel(
        _gather_body,
        out_type=jax.ShapeDtypeStruct((N_PINS,), jnp.float32),
        mesh=plsc.VectorSubcoreMesh(
            core_axis_name="c", subcore_axis_name="s", num_cores=NC,
            num_subcores=NS,
        ),
        scratch_types=[
            pltpu.VMEM((CHUNK,), jnp.int32),
            pltpu.VMEM((CHUNK,), jnp.int32),
            pltpu.VMEM((CHUNK,), jnp.int32),
            pltpu.VMEM((CHUNK,), jnp.int32),
            pltpu.VMEM((CHUNK,), jnp.float32),
            pltpu.VMEM((CHUNK,), jnp.float32),
            pltpu.VMEM((SUB,), jnp.float32),
            pltpu.VMEM((SUB + 16,), jnp.int32),
            pltpu.VMEM((SUB,), jnp.float32),
            pltpu.VMEM_SHARED((NET_PAD,), jnp.float32),
            pltpu.SemaphoreType.DMA,
            pltpu.SemaphoreType.DMA,
            pltpu.SemaphoreType.DMA,
            pltpu.SemaphoreType.DMA,
            pltpu.SemaphoreType.DMA,
        ],
    )


ROWS = N_PINS // 128
RBLK = 5000
NBLK = ROWS // RBLK


def _scan_body(x_ref, y_ref, carry_ref):
    @pl.when(pl.program_id(0) == 0)
    def _():
        carry_ref[0, 0] = 0.0

    x = x_ref[...]
    r = lax.broadcasted_iota(jnp.int32, (128, 128), 0)
    col = lax.broadcasted_iota(jnp.int32, (128, 128), 1)
    tri = (r <= col).astype(jnp.float32)
    y = jnp.dot(x, tri, preferred_element_type=jnp.float32)
    t = y[:, 127:128]
    e = jnp.concatenate([jnp.zeros((1, 1), jnp.float32), t[:-1, :]], axis=0)
    k = 1
    while k < RBLK:
        e = e + jnp.concatenate(
            [jnp.zeros((k, 1), jnp.float32), e[:-k, :]], axis=0
        )
        k *= 2
    c = carry_ref[0, 0]
    y_ref[...] = y + e + c
    carry_ref[0, 0] = c + jnp.sum(t[RBLK - 1:, :]) + jnp.sum(e[RBLK - 1:, :])


def _scan(val2d):
    return pl.pallas_call(
        _scan_body,
        grid=(NBLK,),
        in_specs=[pl.BlockSpec((RBLK, 128), lambda i: (i, 0))],
        out_specs=pl.BlockSpec((RBLK, 128), lambda i: (i, 0)),
        out_shape=jax.ShapeDtypeStruct((ROWS, 128), jnp.float32),
        scratch_shapes=[pltpu.SMEM((1, 1), jnp.float32)],
    )(val2d)


NODES_PER_W = 3136
OUT_PAD = NW * NODES_PER_W
SPAD = NODES_PER_W + 16
START_PAD = (NW - 1) * NODES_PER_W + SPAD


def _bound_body(s_hbm, st_hbm, out_hbm, sv_ref, idx_ref, g_ref, ob_ref, sem):
    wid = lax.axis_index("c") * NS + lax.axis_index("s")
    lo = wid * NODES_PER_W
    pltpu.sync_copy(st_hbm.at[pl.ds(lo, SPAD)], sv_ref)

    def mk_idx(k, carry):
        sv = sv_ref[pl.ds(k * 16, 16)]
        idx_ref[pl.ds(k * 16, 16)] = jnp.maximum(sv - 1, 0)
        return carry

    lax.fori_loop(0, SPAD // 16, mk_idx, 0)
    pltpu.async_copy(s_hbm.at[idx_ref], g_ref, sem).wait()
    iot = lax.iota(jnp.int32, 16)

    def diff(k, carry):
        sv_a = sv_ref[pl.ds(k * 16, 16)]
        sv_b = sv_ref[pl.ds(k * 16 + 1, 16)]
        g_a = g_ref[pl.ds(k * 16, 16)]
        g_b = g_ref[pl.ds(k * 16 + 1, 16)]
        a = jnp.where(sv_a == 0, 0.0, g_a)
        b = jnp.where(sv_b == 0, 0.0, g_b)
        node = lo + k * 16 + iot
        ob_ref[pl.ds(k * 16, 16)] = jnp.where(node < N_MOVABLE, b - a, 0.0)
        return carry

    lax.fori_loop(0, NODES_PER_W // 16, diff, 0)
    pltpu.sync_copy(ob_ref, out_hbm.at[pl.ds(lo, NODES_PER_W)])


def _build_bound():
    return pl.kernel(
        _bound_body,
        out_type=jax.ShapeDtypeStruct((OUT_PAD,), jnp.float32),
        mesh=plsc.VectorSubcoreMesh(
            core_axis_name="c", subcore_axis_name="s", num_cores=NC,
            num_subcores=NS,
        ),
        scratch_types=[
            pltpu.VMEM((SPAD,), jnp.int32),
            pltpu.VMEM((SPAD,), jnp.int32),
            pltpu.VMEM((SPAD,), jnp.float32),
            pltpu.VMEM((NODES_PER_W,), jnp.float32),
            pltpu.SemaphoreType.DMA,
        ],
    )


def kernel(net_weights, flat_node2pin_start, flat_node2pin, pin2net_map,
           flat_net2pin):
    w_pad = jnp.pad(net_weights, (0, NET_PAD - N_NETS))
    f2p_pad = jnp.pad(flat_net2pin, (0, F2P_PAD - (N_NETS + 1)))
    val = _build_gather()(flat_node2pin, pin2net_map, w_pad, f2p_pad)
    s = _scan(val.reshape(ROWS, 128)).reshape(-1)
    stp = jnp.pad(flat_node2pin_start, (0, START_PAD - (N_NODES + 1)))
    return _build_bound()(s, stp)[:N_NODES]
